# Initial kernel scaffold; baseline (speedup 1.0000x reference)
import jax
import jax.numpy as jnp
from jax.experimental import pallas as pl


def kernel(x, masking_matrix, sr_weight, gamma, beta):
    raise NotImplementedError("write your pallas kernel here")


# speedup vs baseline: 849.1290x; 849.1290x over previous
import jax
import jax.numpy as jnp
from jax.experimental import pallas as pl

_B, _N, _D = 8, 192, 196
_INV_SQRT2 = 0.7071067811865476


def _gtn_body(mask_ref, sw_ref, x_ref, gamma_ref, beta_ref, out_ref):
    n = _N
    m = (mask_ref[...] > 0.0).astype(jnp.float32)
    deg = jnp.sum(m, axis=0)
    cnt = jnp.maximum(deg, 1.0)
    inv2 = 1.0 / (cnt * cnt)
    sw = jax.nn.sigmoid(sw_ref[0, 0])
    jj = jax.lax.broadcasted_iota(jnp.int32, (n, n), 0)
    ii = jax.lax.broadcasted_iota(jnp.int32, (n, n), 1)
    scale = jnp.where(jj == ii, 1.0 + sw, 1.0)
    w = m * scale * inv2[None, :]

    gamma = gamma_ref[...]
    beta = beta_ref[...]
    beta_mean = jnp.mean(beta, axis=1)
    inv_nd = 1.0 / float(n * _D)

    for b in range(_B):
        xb = x_ref[b]
        prop = jax.lax.dot_general(
            w, xb, (((0,), (0,)), ((), ())),
            preferred_element_type=jnp.float32)
        t = prop + xb
        h = 0.5 * t * (1.0 + jax.lax.erf(t * _INV_SQRT2))
        mu = jnp.sum(h) * inv_nd
        hc = h - mu
        var = jnp.sum(hc * hc) * inv_nd
        rs = jax.lax.rsqrt(var + 1e-5)
        out_ref[b, :] = rs * jnp.mean(hc * gamma, axis=1) + beta_mean


def kernel(x, masking_matrix, sr_weight, gamma, beta):
    mm = masking_matrix.reshape(_N, _N)
    sw = sr_weight.reshape(1, 1)
    return pl.pallas_call(
        _gtn_body,
        out_shape=jax.ShapeDtypeStruct((_B, _N), jnp.float32),
    )(mm, sw, x, gamma, beta)


# speedup vs baseline: 981.4708x; 1.1559x over previous
import jax
import jax.numpy as jnp
from jax.experimental import pallas as pl

_B, _N, _D = 8, 192, 196
_INV_SQRT2 = 0.7071067811865476


def _gtn_body(mask_ref, sw_ref, x_ref, gamma_ref, beta_ref, out_ref):
    n, d = _N, _D
    ones_d = jnp.ones((d, 1), jnp.float32)
    ones_n = jnp.ones((n, 1), jnp.float32)

    def colsum(a):
        return jax.lax.dot_general(
            a, ones_d if a.shape[1] == d else ones_n,
            (((1,), (0,)), ((), ())), preferred_element_type=jnp.float32)

    m = (mask_ref[...] > 0.0).astype(jnp.float32)
    mt = m.T
    deg = colsum(mt)
    cnt = jnp.maximum(deg, 1.0)
    inv2 = 1.0 / (cnt * cnt)
    sw = jax.nn.sigmoid(sw_ref[0, 0])
    ii = jax.lax.broadcasted_iota(jnp.int32, (n, n), 0)
    jj = jax.lax.broadcasted_iota(jnp.int32, (n, n), 1)
    scale = jnp.where(ii == jj, 1.0 + sw, 1.0)
    wt = mt * scale * inv2

    gamma = gamma_ref[...]
    beta = beta_ref[...]
    inv_d = 1.0 / float(d)
    inv_nd = 1.0 / float(n * d)
    gm = colsum(gamma) * inv_d
    bm = colsum(beta) * inv_d

    cols = []
    for b in range(_B):
        xb = x_ref[b]
        prop = jax.lax.dot_general(
            wt, xb, (((1,), (0,)), ((), ())),
            preferred_element_type=jnp.float32)
        t = prop + xb
        h = 0.5 * t * (1.0 + jax.lax.erf(t * _INV_SQRT2))
        s_hg = colsum(h * gamma)
        mu = jnp.sum(h) * inv_nd
        var = jnp.sum(h * h) * inv_nd - mu * mu
        rs = jax.lax.rsqrt(var + 1e-5)
        cols.append(rs * (s_hg * inv_d - mu * gm) + bm)

    out_ref[...] = jnp.concatenate(cols, axis=1).T


def kernel(x, masking_matrix, sr_weight, gamma, beta):
    mm = masking_matrix.reshape(_N, _N)
    sw = sr_weight.reshape(1, 1)
    return pl.pallas_call(
        _gtn_body,
        out_shape=jax.ShapeDtypeStruct((_B, _N), jnp.float32),
    )(mm, sw, x, gamma, beta)


# speedup vs baseline: 1259.3585x; 1.2831x over previous
import numpy as np
import jax
import jax.numpy as jnp
from jax.experimental import pallas as pl

_B, _N, _D = 8, 192, 196
_INV_SQRT2 = 0.7071067811865476

_p = np.arange(_N)[:, None]
_s = np.arange(288)[None, :]
_base = 3 * (_p // 2) + (_p % 2)
_SEL_A = (_s == _base).astype(np.float32)
_SEL_B = (_s == _base + 1).astype(np.float32)


def _gtn_body(mask_ref, sel_a_ref, sel_b_ref, sw_ref, x_ref, gamma_ref,
              beta_ref, out_ref):
    n, d = _N, _D
    ones_d = jnp.ones((d, 1), jnp.float32)
    ones_n = jnp.ones((n, 1), jnp.float32)

    def colsum(a):
        return jax.lax.dot_general(
            a, ones_d if a.shape[1] == d else ones_n,
            (((1,), (0,)), ((), ())), preferred_element_type=jnp.float32)

    def selmul(sel, rhs):
        return jax.lax.dot_general(
            sel, rhs, (((1,), (0,)), ((), ())),
            preferred_element_type=jnp.float32)

    m288 = (mask_ref[...] > 0.0).astype(jnp.bfloat16)
    u = selmul(sel_a_ref[...], m288)
    v = selmul(sel_b_ref[...], m288)
    m_even = jnp.concatenate([u, v[:, :64]], axis=1)
    m_odd = jnp.concatenate([u[:, 64:], v], axis=1)
    par = jax.lax.broadcasted_iota(jnp.int32, (n, n), 0) % 2
    m = jnp.where(par == 0, m_even, m_odd)

    mt = m.T
    deg = colsum(mt)
    cnt = jnp.maximum(deg, 1.0)
    inv2 = 1.0 / (cnt * cnt)
    sw = jax.nn.sigmoid(sw_ref[0, 0])
    ii = jax.lax.broadcasted_iota(jnp.int32, (n, n), 0)
    jj = jax.lax.broadcasted_iota(jnp.int32, (n, n), 1)
    scale = jnp.where(ii == jj, 1.0 + sw, 1.0)
    wt = mt * scale * inv2

    gamma = gamma_ref[...]
    beta = beta_ref[...]
    inv_d = 1.0 / float(d)
    inv_nd = 1.0 / float(n * d)
    gm = colsum(gamma) * inv_d
    bm = colsum(beta) * inv_d

    cols = []
    for b in range(_B):
        xb = x_ref[b]
        prop = jax.lax.dot_general(
            wt, xb, (((1,), (0,)), ((), ())),
            preferred_element_type=jnp.float32)
        t = prop + xb
        h = 0.5 * t * (1.0 + jax.lax.erf(t * _INV_SQRT2))
        s_hg = colsum(h * gamma)
        mu = jnp.sum(h) * inv_nd
        var = jnp.sum(h * h) * inv_nd - mu * mu
        rs = jax.lax.rsqrt(var + 1e-5)
        cols.append(rs * (s_hg * inv_d - mu * gm) + bm)

    out_ref[...] = jnp.concatenate(cols, axis=1).T


def kernel(x, masking_matrix, sr_weight, gamma, beta):
    mm = masking_matrix.reshape(288, 128)
    sw = sr_weight.reshape(1, 1)
    sel_a = jnp.asarray(_SEL_A, dtype=jnp.bfloat16)
    sel_b = jnp.asarray(_SEL_B, dtype=jnp.bfloat16)
    return pl.pallas_call(
        _gtn_body,
        out_shape=jax.ShapeDtypeStruct((_B, _N), jnp.float32),
    )(mm, sel_a, sel_b, sw, x, gamma, beta)
